# Initial kernel scaffold; baseline (speedup 1.0000x reference)
import jax
import jax.numpy as jnp
from jax.experimental import pallas as pl


def kernel(relation_ids, timestamps, numeric_values, string_hashes, type_indicators, relation_table, abs_tab, rel_tab, sess_tab, Wp, bp, Wn, bn, string_tab, type_tab, W1, b1, W2, b2, pe):
    raise NotImplementedError("write your pallas kernel here")


# speedup vs baseline: 7.3576x; 7.3576x over previous
import functools
import math

import jax
import jax.numpy as jnp
from jax import lax
from jax.experimental import pallas as pl
from jax.experimental.pallas import tpu as pltpu
from jax.experimental.pallas import tpu_sc as plsc

B, S, D = 1024, 50, 128
N = B * S
V = 100000
NB = 100
MV = 10
HID = 128
D3 = D // 3

NC, NS = 2, 16
NW = NC * NS
TPW = N // NW
CH = 64
NCHUNK = TPW // CH
ROWS = CH * MV
NGS = ROWS // 128


def _sc_gather(rel_idx, str_idx, rel_table, str_table, zeros, scidx_all):
    mesh = plsc.VectorSubcoreMesh(core_axis_name="c", subcore_axis_name="s",
                                  num_cores=NC, num_subcores=NS)

    @functools.partial(
        pl.kernel,
        out_type=(jax.ShapeDtypeStruct((N, D), jnp.float32),
                  jax.ShapeDtypeStruct((N, D), jnp.float32)),
        mesh=mesh,
        scratch_types=[
            pltpu.VMEM((CH,), jnp.int32),
            pltpu.VMEM((CH, D), jnp.float32),
            pltpu.VMEM((ROWS,), jnp.int32),
            pltpu.VMEM((ROWS, D), jnp.float32),
            pltpu.VMEM((NGS, 128), jnp.int32),
            pltpu.VMEM((CH, D), jnp.float32),
            pltpu.VMEM_SHARED((NS * CH, D), jnp.float32),
            pltpu.SemaphoreType.DMA,
            pltpu.SemaphoreType.DMA,
        ],
    )
    def k(rel_idx_hbm, str_idx_hbm, rel_tab_hbm, str_tab_hbm, zeros_hbm,
          scidx_hbm, rel_out, str_out,
          ridx_v, rrows_v, sidx_v, srows_v, scidx_v, zero_v, acc_sh,
          rsem, ssem):
        cid = lax.axis_index("c")
        sid = lax.axis_index("s")
        wid = sid * NC + cid
        base0 = wid * TPW

        pltpu.sync_copy(zeros_hbm, zero_v)
        pltpu.sync_copy(scidx_hbm.at[sid], scidx_v)

        def chunk(c, carry):
            base = base0 + c * CH
            pltpu.sync_copy(rel_idx_hbm.at[pl.ds(base, CH)], ridx_v)
            rdma = pltpu.async_copy(rel_tab_hbm.at[ridx_v], rrows_v, rsem)
            pltpu.sync_copy(str_idx_hbm.at[pl.ds(base * MV, ROWS)], sidx_v)
            sdmas = [
                pltpu.async_copy(
                    str_tab_hbm.at[sidx_v.at[pl.ds(j * 128, 128)]],
                    srows_v.at[pl.ds(j * 128, 128)], ssem)
                for j in range(NGS)
            ]
            pltpu.sync_copy(zero_v, acc_sh.at[pl.ds(sid * CH, CH)])
            rdma.wait()
            pltpu.sync_copy(rrows_v, rel_out.at[pl.ds(base, CH)])
            for j in range(NGS):
                sdmas[j].wait()
                pltpu.sync_copy(srows_v.at[pl.ds(j * 128, 128)],
                                acc_sh.at[scidx_v.at[j]], add=True)
            pltpu.sync_copy(acc_sh.at[pl.ds(sid * CH, CH)],
                            str_out.at[pl.ds(base, CH)])
            return carry

        lax.fori_loop(0, NCHUNK, chunk, 0)

    return k(rel_idx, str_idx, rel_table, str_table, zeros, scidx_all)


TBLK = 400
GRID = N // TBLK

_LOG_MAX = math.log(1e6)


def _quant(x):
    c = jnp.maximum(x, 1.0)
    lt = jnp.log(c) / _LOG_MAX * (NB - 1)
    return jnp.clip(lt.astype(jnp.int32), 0, NB - 1)


def _tc_body(ts_ref, rt_ref, st_ref, num_ref, tind_ref, rel_ref, ssum_ref,
             A_ref, R_ref, Se_ref, T4_ref, Wna_ref, W1b_ref, W2_ref,
             b1_ref, fb_ref, pe_ref, out_ref):
    f32 = jnp.float32
    qa = _quant(ts_ref[...])
    qr = _quant(rt_ref[...])
    qs = _quant(st_ref[...])
    ioc = lax.broadcasted_iota(jnp.int32, (1, NB), 1)
    oha = (qa == ioc).astype(f32)
    ohr = (qr == ioc).astype(f32)
    ohs = (qs == ioc).astype(f32)
    dot = functools.partial(jnp.dot, preferred_element_type=f32)
    temporal = dot(oha, A_ref[...]) + dot(ohr, R_ref[...]) + dot(ohs, Se_ref[...])
    io4 = lax.broadcasted_iota(jnp.int32, (1, 4), 1)
    oht = (tind_ref[...] == io4).astype(f32)
    pre = (dot(num_ref[...], Wna_ref[...])
           + dot(ssum_ref[...] * (1.0 / MV), W1b_ref[...])
           + dot(oht, T4_ref[...]) + b1_ref[...])
    val = dot(jnp.maximum(pre, 0.0), W2_ref[...])
    out_ref[...] = rel_ref[...] + temporal + val + pe_ref[...] + fb_ref[...]


def _tc_call(ts, rt, st, xnum, tind, rel_rows, str_sum,
             A, R, Se, T4, Wna, W1b, W2, b1f, fb, pe_t):
    blk = lambda r, c: pl.BlockSpec((TBLK, c), lambda i: (i, 0))
    full = lambda a, b: pl.BlockSpec((a, b), lambda i: (0, 0))
    return pl.pallas_call(
        _tc_body,
        grid=(GRID,),
        in_specs=[
            blk(TBLK, 1), blk(TBLK, 1), blk(TBLK, 1),
            blk(TBLK, MV), blk(TBLK, 1),
            blk(TBLK, D), blk(TBLK, D),
            full(NB, D), full(NB, D), full(NB, D),
            full(4, D), full(MV, D), full(HID, D), full(D, D),
            full(1, D), full(1, D), full(TBLK, D),
        ],
        out_specs=pl.BlockSpec((TBLK, D), lambda i: (i, 0)),
        out_shape=jax.ShapeDtypeStruct((N, D), jnp.float32),
    )(ts, rt, st, xnum, tind, rel_rows, str_sum,
      A, R, Se, T4, Wna, W1b, W2, b1f, fb, pe_t)


def kernel(relation_ids, timestamps, numeric_values, string_hashes,
           type_indicators, relation_table, abs_tab, rel_tab, sess_tab,
           Wp, bp, Wn, bn, string_tab, type_tab, W1, b1, W2, b2, pe):
    f32 = jnp.float32
    ts = timestamps.astype(f32)
    rel_t = jnp.concatenate(
        [jnp.zeros_like(ts[:, :1]), ts[:, 1:] - ts[:, :-1]], axis=1)
    sess_t = ts - ts[:, :1]

    ridx = relation_ids.reshape(N).astype(jnp.int32)
    sidx = string_hashes.reshape(N * MV).astype(jnp.int32)
    zeros = jnp.zeros((CH, D), f32)
    r_ids = jnp.arange(NGS * 128, dtype=jnp.int32) // MV
    scidx_all = (jnp.arange(NS, dtype=jnp.int32)[:, None] * CH
                 + r_ids[None, :]).reshape(NS, NGS, 128)
    rel_rows, str_sum = _sc_gather(ridx, sidx, relation_table.astype(f32),
                                   string_tab.astype(f32), zeros, scidx_all)

    A = abs_tab @ Wp[:D3]
    R = rel_tab @ Wp[D3:2 * D3]
    Se = sess_tab @ Wp[2 * D3:]
    W1a, W1b, W1c = W1[:HID], W1[HID:2 * HID], W1[2 * HID:]
    Wna = Wn @ W1a
    T4 = type_tab @ W1c
    b1f = (b1 + bn @ W1a).reshape(1, D)
    fb = (bp + b2).reshape(1, D)
    pe_t = jnp.tile(pe[:S], (TBLK // S, 1))

    out = _tc_call(ts.reshape(N, 1), rel_t.reshape(N, 1), sess_t.reshape(N, 1),
                   numeric_values.reshape(N, MV), type_indicators.reshape(N, 1).astype(jnp.int32),
                   rel_rows, str_sum,
                   A, R, Se, T4, Wna, W1b, W2, b1f, fb, pe_t)
    return out.reshape(B, S, D)


# speedup vs baseline: 7.9049x; 1.0744x over previous
import functools
import math

import jax
import jax.numpy as jnp
from jax import lax
from jax.experimental import pallas as pl
from jax.experimental.pallas import tpu as pltpu
from jax.experimental.pallas import tpu_sc as plsc

B, S, D = 1024, 50, 128
N = B * S
V = 100000
NB = 100
MV = 10
HID = 128
D3 = D // 3

NC, NS = 2, 16
NW = NC * NS
TPW = N // NW
CH = 32
NCHUNK = TPW // CH
ROWS = CH * MV
GSZ = 64
NGS = ROWS // GSZ


def _sc_gather(rel_idx, str_idx, rel_table, str_table, zeros, scidx_all):
    mesh = plsc.VectorSubcoreMesh(core_axis_name="c", subcore_axis_name="s",
                                  num_cores=NC, num_subcores=NS)

    @functools.partial(
        pl.kernel,
        out_type=(jax.ShapeDtypeStruct((N, D), jnp.float32),
                  jax.ShapeDtypeStruct((N, D), jnp.float32)),
        mesh=mesh,
        scratch_types=[
            pltpu.VMEM((TPW,), jnp.int32),
            pltpu.VMEM((TPW * MV,), jnp.int32),
            pltpu.VMEM((CH, D), jnp.float32),
            pltpu.VMEM((CH, D), jnp.float32),
            pltpu.VMEM((ROWS, D), jnp.float32),
            pltpu.VMEM((ROWS, D), jnp.float32),
            pltpu.VMEM((2, NGS, GSZ), jnp.int32),
            pltpu.VMEM((CH, D), jnp.float32),
            pltpu.VMEM_SHARED((NS * 2 * CH, D), jnp.float32),
            pltpu.SemaphoreType.DMA, pltpu.SemaphoreType.DMA,
            pltpu.SemaphoreType.DMA, pltpu.SemaphoreType.DMA,
            pltpu.SemaphoreType.DMA, pltpu.SemaphoreType.DMA,
        ],
    )
    def k(rel_idx_hbm, str_idx_hbm, rel_tab_hbm, str_tab_hbm, zeros_hbm,
          scidx_hbm, rel_out, str_out,
          ridx_all, sidx_all, rrow0, rrow1, srow0, srow1, scidx_v, zero_v,
          acc_sh, rsem0, rsem1, ssem0, ssem1, asem0, asem1):
        cid = lax.axis_index("c")
        sid = lax.axis_index("s")
        wid = sid * NC + cid
        base0 = wid * TPW

        rrow = (rrow0, rrow1)
        srow = (srow0, srow1)
        rsem = (rsem0, rsem1)
        ssem = (ssem0, ssem1)
        asem = (asem0, asem1)

        pltpu.sync_copy(zeros_hbm, zero_v)
        pltpu.sync_copy(scidx_hbm.at[sid], scidx_v)
        pltpu.sync_copy(rel_idx_hbm.at[pl.ds(base0, TPW)], ridx_all)
        pltpu.sync_copy(str_idx_hbm.at[pl.ds(base0 * MV, TPW * MV)], sidx_all)

        def rel_gather_args(c, b):
            off = c * CH
            return (rel_tab_hbm.at[ridx_all.at[pl.ds(off, CH)]],
                    rrow[b], rsem[b])

        def str_gather_args(c, b, j):
            off = c * ROWS + j * GSZ
            return (str_tab_hbm.at[sidx_all.at[pl.ds(off, GSZ)]],
                    srow[b].at[pl.ds(j * GSZ, GSZ)], ssem[b])

        def add_args(b, j):
            return (srow[b].at[pl.ds(j * GSZ, GSZ)],
                    acc_sh.at[scidx_v.at[b, j]], asem[b])

        def issue(c, b):
            pltpu.async_copy(*rel_gather_args(c, b))
            for j in range(NGS):
                pltpu.async_copy(*str_gather_args(c, b, j))

        def process(c, b):
            base = base0 + c * CH
            acc_region = acc_sh.at[pl.ds((sid * 2 + b) * CH, CH)]
            pltpu.sync_copy(zero_v, acc_region)
            pltpu.make_async_copy(*rel_gather_args(c, b)).wait()
            pltpu.sync_copy(rrow[b], rel_out.at[pl.ds(base, CH)])
            for j in range(NGS):
                pltpu.make_async_copy(*str_gather_args(c, b, j)).wait()
            adds = [pltpu.async_copy(*add_args(b, j), add=True)
                    for j in range(NGS)]
            for a in adds:
                a.wait()
            pltpu.sync_copy(acc_region, str_out.at[pl.ds(base, CH)])

        issue(0, 0)
        issue(1, 1)

        def body(cc, carry):
            c = cc * 2
            process(c, 0)
            issue(c + 2, 0)
            process(c + 1, 1)
            issue(c + 3, 1)
            return carry

        lax.fori_loop(0, NCHUNK // 2 - 1, body, 0)
        process(NCHUNK - 2, 0)
        process(NCHUNK - 1, 1)

    return k(rel_idx, str_idx, rel_table, str_table, zeros, scidx_all)


TBLK = 400
GRID = N // TBLK

_LOG_MAX = math.log(1e6)


def _quant(x):
    c = jnp.maximum(x, 1.0)
    lt = jnp.log(c) / _LOG_MAX * (NB - 1)
    return jnp.clip(lt.astype(jnp.int32), 0, NB - 1)


def _tc_body(ts_ref, rt_ref, st_ref, num_ref, tind_ref, rel_ref, ssum_ref,
             A_ref, R_ref, Se_ref, T4_ref, Wna_ref, W1b_ref, W2_ref,
             b1_ref, fb_ref, pe_ref, out_ref):
    f32 = jnp.float32
    qa = _quant(ts_ref[...])
    qr = _quant(rt_ref[...])
    qs = _quant(st_ref[...])
    ioc = lax.broadcasted_iota(jnp.int32, (1, NB), 1)
    oha = (qa == ioc).astype(f32)
    ohr = (qr == ioc).astype(f32)
    ohs = (qs == ioc).astype(f32)
    dot = functools.partial(jnp.dot, preferred_element_type=f32)
    temporal = dot(oha, A_ref[...]) + dot(ohr, R_ref[...]) + dot(ohs, Se_ref[...])
    io4 = lax.broadcasted_iota(jnp.int32, (1, 4), 1)
    oht = (tind_ref[...] == io4).astype(f32)
    pre = (dot(num_ref[...], Wna_ref[...])
           + dot(ssum_ref[...] * (1.0 / MV), W1b_ref[...])
           + dot(oht, T4_ref[...]) + b1_ref[...])
    val = dot(jnp.maximum(pre, 0.0), W2_ref[...])
    out_ref[...] = rel_ref[...] + temporal + val + pe_ref[...] + fb_ref[...]


def _tc_call(ts, rt, st, xnum, tind, rel_rows, str_sum,
             A, R, Se, T4, Wna, W1b, W2, b1f, fb, pe_t):
    blk = lambda r, c: pl.BlockSpec((TBLK, c), lambda i: (i, 0))
    full = lambda a, b: pl.BlockSpec((a, b), lambda i: (0, 0))
    return pl.pallas_call(
        _tc_body,
        grid=(GRID,),
        in_specs=[
            blk(TBLK, 1), blk(TBLK, 1), blk(TBLK, 1),
            blk(TBLK, MV), blk(TBLK, 1),
            blk(TBLK, D), blk(TBLK, D),
            full(NB, D), full(NB, D), full(NB, D),
            full(4, D), full(MV, D), full(HID, D), full(D, D),
            full(1, D), full(1, D), full(TBLK, D),
        ],
        out_specs=pl.BlockSpec((TBLK, D), lambda i: (i, 0)),
        out_shape=jax.ShapeDtypeStruct((N, D), jnp.float32),
    )(ts, rt, st, xnum, tind, rel_rows, str_sum,
      A, R, Se, T4, Wna, W1b, W2, b1f, fb, pe_t)


def kernel(relation_ids, timestamps, numeric_values, string_hashes,
           type_indicators, relation_table, abs_tab, rel_tab, sess_tab,
           Wp, bp, Wn, bn, string_tab, type_tab, W1, b1, W2, b2, pe):
    f32 = jnp.float32
    ts = timestamps.astype(f32)
    rel_t = jnp.concatenate(
        [jnp.zeros_like(ts[:, :1]), ts[:, 1:] - ts[:, :-1]], axis=1)
    sess_t = ts - ts[:, :1]

    ridx = relation_ids.reshape(N).astype(jnp.int32)
    sidx = string_hashes.reshape(N * MV).astype(jnp.int32)
    zeros = jnp.zeros((CH, D), f32)
    r_ids = jnp.arange(ROWS, dtype=jnp.int32) // MV
    slot = jnp.arange(2, dtype=jnp.int32)[:, None] * CH
    sidb = jnp.arange(NS, dtype=jnp.int32)[:, None, None] * (2 * CH)
    scidx_all = (sidb + slot[None] + r_ids[None, None, :]
                 ).reshape(NS, 2, NGS, GSZ)
    rel_rows, str_sum = _sc_gather(ridx, sidx, relation_table.astype(f32),
                                   string_tab.astype(f32), zeros, scidx_all)

    A = abs_tab @ Wp[:D3]
    R = rel_tab @ Wp[D3:2 * D3]
    Se = sess_tab @ Wp[2 * D3:]
    W1a, W1b, W1c = W1[:HID], W1[HID:2 * HID], W1[2 * HID:]
    Wna = Wn @ W1a
    T4 = type_tab @ W1c
    b1f = (b1 + bn @ W1a).reshape(1, D)
    fb = (bp + b2).reshape(1, D)
    pe_t = jnp.tile(pe[:S], (TBLK // S, 1))

    out = _tc_call(ts.reshape(N, 1), rel_t.reshape(N, 1), sess_t.reshape(N, 1),
                   numeric_values.reshape(N, MV), type_indicators.reshape(N, 1).astype(jnp.int32),
                   rel_rows, str_sum,
                   A, R, Se, T4, Wna, W1b, W2, b1f, fb, pe_t)
    return out.reshape(B, S, D)
